# baseline (speedup 1.0000x reference)
import functools

import jax
import jax.numpy as jnp
from jax import lax
from jax.experimental import pallas as pl
from jax.experimental.pallas import tpu as pltpu
from jax.experimental.pallas import tpu_sc as plsc

_NC = 2
_NS = 16
_NW = _NC * _NS
_IDX = 128
_MB = 256
_RING = 3
_S = 512


def _sc_body(feat_hbm, ids_hbm, out_hbm, ids_v, bufs, zbuf, acc,
             sem0, sem1, sem2,
             *, total_mb, qsb, extra_sb, rem, nb, d):
    c = lax.axis_index("c")
    s = lax.axis_index("s")
    wid = c * _NS + s
    nblk = 4 * (qsb + (wid < extra_sb).astype(jnp.int32)) \
        + (wid == _NW - 1).astype(jnp.int32) * rem
    start = 4 * (wid * qsb + jnp.minimum(wid, extra_sb))

    zero = jnp.zeros((16,), jnp.float32)
    for r in range(_S // _NS):
        for k8 in range(d // 16):
            zbuf[r, pl.ds(k8 * 16, 16)] = zero
    pltpu.sync_copy(zbuf, acc.at[pl.ds(s * (_S // _NS), _S // _NS)])

    idw = (nb * (_MB // _IDX) + 7) // 8 * 8
    pltpu.sync_copy(ids_hbm.at[pl.ds(start * (_MB // _IDX), idw)], ids_v)

    def _blk_slice(i):
        bi = jnp.minimum(start + i, total_mb - 1)
        return feat_hbm.at[pl.ds(bi * _MB, _MB)]

    sems = (sem0, sem1, sem2)
    pltpu.async_copy(_blk_slice(0), bufs.at[0], sems[0])
    pltpu.async_copy(_blk_slice(1), bufs.at[1], sems[1])

    plsc.subcore_barrier()

    def _outer(g, carry):
        for b in range(_RING):
            i = _RING * g + b
            pltpu.make_async_copy(_blk_slice(0), bufs.at[b], sems[b]).wait()

            @pl.when(i + 2 < nb)
            def _refill():
                b2 = (b + 2) % _RING
                pltpu.async_copy(_blk_slice(i + 2), bufs.at[b2], sems[b2])

            @pl.when(i < nblk)
            def _scatter():
                for h in range(_MB // _IDX):
                    pltpu.sync_copy(bufs.at[b, pl.ds(h * _IDX, _IDX)],
                                    acc.at[ids_v.at[i * (_MB // _IDX) + h]],
                                    add=True)
        return carry

    lax.fori_loop(0, nb // _RING, _outer, 0)

    plsc.subcore_barrier()
    rows = _S // _NS
    pltpu.sync_copy(acc.at[pl.ds(s * rows, rows)],
                    out_hbm.at[pl.ds(c * _S + s * rows, rows)])


def _combine_body(p_ref, o_ref):
    o_ref[...] = p_ref[:_S, :] + p_ref[_S:, :]


def kernel(feat, segment_ids, num_segments):
    e, d = feat.shape
    total_mb = e // _MB
    total_sb = total_mb // 4
    rem = total_mb - 4 * total_sb
    qsb = total_sb // _NW
    extra_sb = total_sb - qsb * _NW
    max_blk = max(4 * (qsb + (1 if extra_sb else 0)), 4 * qsb + rem)
    nb = ((max_blk + _RING - 1) // _RING) * _RING

    ids = (segment_ids + (num_segments - _S)).astype(jnp.int32)
    idw = (nb * (_MB // _IDX) + 7) // 8 * 8
    pad_rows = (e // _IDX) + idw
    ids2d = jnp.pad(ids, (0, pad_rows * _IDX - e)).reshape(pad_rows, _IDX)

    mesh = plsc.VectorSubcoreMesh(core_axis_name="c", subcore_axis_name="s",
                                  num_cores=_NC, num_subcores=_NS)
    body = functools.partial(_sc_body, total_mb=total_mb, qsb=qsb,
                             extra_sb=extra_sb, rem=rem, nb=nb, d=d)
    partials = pl.kernel(
        body,
        jax.ShapeDtypeStruct((_NC * _S, d), jnp.float32),
        mesh=mesh,
        scratch_types=[
            pltpu.VMEM((idw, _IDX), jnp.int32),
            pltpu.VMEM((_RING, _MB, d), jnp.float32),
            pltpu.VMEM((_S // _NS, d), jnp.float32),
            pltpu.VMEM_SHARED((_S, d), jnp.float32),
            pltpu.SemaphoreType.DMA,
            pltpu.SemaphoreType.DMA,
            pltpu.SemaphoreType.DMA,
        ],
    )(feat, ids2d)

    return pl.pallas_call(
        _combine_body,
        out_shape=jax.ShapeDtypeStruct((_S, d), jnp.float32),
    )(partials)
